# baseline (speedup 1.0000x reference)
import jax
import jax.numpy as jnp
from jax import lax
from jax.experimental import pallas as pl
from jax.experimental.pallas import tpu as pltpu
from jax.experimental.pallas import tpu_sc as plsc

_INFO = plsc.get_sparse_core_info()
_NC, _NS, _L = _INFO.num_cores, _INFO.num_subcores, _INFO.num_lanes
_NW = _NC * _NS

_D4 = 512
_C = 64
_SUP = 2048
_NCH = _SUP // _C


def _sc_body(b0_hbm, r0_hbm, b1_hbm, r1_hbm, tab_hbm, out_hbm,
             ib0, ir0, ib1, ir1, cidx, rows0, rows1,
             gsem0, gsem1, osem0, osem1):
    cid = lax.axis_index("c")
    sid = lax.axis_index("s")
    wid = sid * _NC + cid
    n_pairs = b0_hbm.shape[0]
    per_w = n_pairs // _NW
    n_super = per_w // _SUP
    tab_off = wid * 400

    rows = (rows0, rows1)
    gsems = (gsem0, gsem1)
    osems = (osem0, osem1)

    def super_body(s_i, _):
        sup_start = wid * per_w + s_i * _SUP
        sl = pl.ds(sup_start, _SUP)
        pltpu.sync_copy(b0_hbm.at[sl], ib0)
        pltpu.sync_copy(r0_hbm.at[sl], ir0)
        pltpu.sync_copy(b1_hbm.at[sl], ib1)
        pltpu.sync_copy(r1_hbm.at[sl], ir1)

        def cvt(i, _):
            s = pl.ds(i * _L, _L)
            cidx[s] = ((ib0[s] * 5 + ir0[s]) * 20
                       + (ib1[s] * 5 + ir1[s]) + tab_off)
            return _

        lax.fori_loop(0, _SUP // _L, cvt, 0)

        def pair_body(p, _):
            for b in range(2):
                ch = p * 2 + b
                first_use = (s_i == 0) & (p == 0)

                @pl.when(jnp.logical_not(first_use))
                def _wait():
                    pltpu.make_async_copy(
                        rows[b], out_hbm.at[pl.ds(0, _C)], osems[b]).wait()

                pltpu.async_copy(
                    tab_hbm.at[cidx.at[pl.ds(ch * _C, _C)]],
                    rows[b], gsems[b]).wait()
                out_off = sup_start + ch * _C
                pltpu.async_copy(
                    rows[b], out_hbm.at[pl.ds(out_off, _C)], osems[b])
            return _

        lax.fori_loop(0, _NCH // 2, pair_body, 0)
        return _

    lax.fori_loop(0, n_super, super_body, 0)

    for b in range(2):
        pltpu.make_async_copy(
            rows[b], out_hbm.at[pl.ds(0, _C)], osems[b]).wait()


@jax.jit
def kernel(batch_base_seq, batch_ref_seq, read_table, ref_table):
    B, S = batch_base_seq.shape
    D = read_table.shape[1]
    N = B * S
    c = jnp.arange(20)
    combined = jnp.concatenate(
        [read_table[c // 5], ref_table[c % 5]], axis=1)
    cp = jnp.arange(400)
    pair_tab = jnp.concatenate(
        [combined[cp // 20], combined[cp % 20]], axis=1)
    rep_tab = jnp.tile(pair_tab, (_NW, 1))
    base = batch_base_seq.astype(jnp.int32).reshape(N // 2, 2)
    refi = batch_ref_seq.astype(jnp.int32).reshape(N // 2, 2)
    b0, b1 = base[:, 0], base[:, 1]
    r0, r1 = refi[:, 0], refi[:, 1]

    run = pl.kernel(
        _sc_body,
        out_type=jax.ShapeDtypeStruct((N // 2, 4 * D), jnp.float32),
        mesh=plsc.VectorSubcoreMesh(core_axis_name="c", subcore_axis_name="s"),
        scratch_types=[
            pltpu.VMEM((_SUP,), jnp.int32),
            pltpu.VMEM((_SUP,), jnp.int32),
            pltpu.VMEM((_SUP,), jnp.int32),
            pltpu.VMEM((_SUP,), jnp.int32),
            pltpu.VMEM((_SUP,), jnp.int32),
            pltpu.VMEM((_C, _D4), jnp.float32),
            pltpu.VMEM((_C, _D4), jnp.float32),
            pltpu.SemaphoreType.DMA,
            pltpu.SemaphoreType.DMA,
            pltpu.SemaphoreType.DMA,
            pltpu.SemaphoreType.DMA,
        ],
    )
    out = run(b0, r0, b1, r1, rep_tab)
    return out.reshape(B, S, 2 * D)
